# baseline (speedup 1.0000x reference)
import functools

import jax
import jax.numpy as jnp
from jax import lax
from jax.experimental import pallas as pl
from jax.experimental.pallas import tpu as pltpu
from jax.experimental.pallas import tpu_sc as plsc

NC = 2
NS = 16
NW = NC * NS
CHUNK = 128
SPLIT0_NUM, SPLIT_DEN = 5, 10
DW = 16
ROWBLK = 1024


def _sc_mesh():
    return plsc.VectorSubcoreMesh(core_axis_name="c", subcore_axis_name="s")


def _make_deg(CH0, CH1, CHM, NP):
    rpt = NP // NS

    @functools.partial(
        pl.kernel,
        out_type=jax.ShapeDtypeStruct((NC, NP, DW), jnp.float32),
        mesh=_sc_mesh(),
        compiler_params=pltpu.CompilerParams(use_tc_tiling_on_sc=False),
        scratch_types=[
            pltpu.VMEM((CHM, CHUNK), jnp.int32),
            pltpu.VMEM((CHUNK, DW), jnp.float32),
            pltpu.VMEM_SHARED((NP, DW), jnp.float32),
        ],
    )
    def deg_kernel(dst_hbm, ones_hbm, zeros_hbm, out_hbm, idx_d, ones_v, acc):
        c = lax.axis_index("c")
        s = lax.axis_index("s")
        wid = s * NC + c
        pltpu.sync_copy(zeros_hbm.at[pl.ds(s * rpt, rpt)],
                        acc.at[pl.ds(s * rpt, rpt)])
        pltpu.sync_copy(dst_hbm.at[wid], idx_d)
        pltpu.sync_copy(ones_hbm, ones_v)
        plsc.subcore_barrier()

        def body(j, carry):
            pltpu.sync_copy(ones_v, acc.at[idx_d.at[j]], add=True)
            return carry

        @pl.when(c == 0)
        def _c0():
            lax.fori_loop(0, CH0, body, 0)

        @pl.when(c != 0)
        def _c1():
            lax.fori_loop(0, CH1, body, 0)

        plsc.subcore_barrier()
        pltpu.sync_copy(acc.at[pl.ds(s * rpt, rpt)],
                        out_hbm.at[c, pl.ds(s * rpt, rpt)])

    return deg_kernel


def _make_prop(CH0, CH1, CHM, NP, H):
    rpt = NP // NS

    @functools.partial(
        pl.kernel,
        out_type=jax.ShapeDtypeStruct((NC, NP, H), jnp.float32),
        mesh=_sc_mesh(),
        compiler_params=pltpu.CompilerParams(use_tc_tiling_on_sc=False),
        scratch_types=[
            pltpu.VMEM((CHM, CHUNK), jnp.int32),
            pltpu.VMEM((CHM, CHUNK), jnp.int32),
            pltpu.VMEM((CHUNK, H), jnp.float32),
            pltpu.SemaphoreType.DMA,
            pltpu.VMEM_SHARED((NP, H), jnp.float32),
        ],
    )
    def prop_kernel(src_hbm, dst_hbm, f_hbm, zeros_hbm, out_hbm,
                    idx_s, idx_d, rows, sem, acc):
        c = lax.axis_index("c")
        s = lax.axis_index("s")
        wid = s * NC + c
        pltpu.sync_copy(zeros_hbm.at[pl.ds(s * rpt, rpt)],
                        acc.at[pl.ds(s * rpt, rpt)])
        pltpu.sync_copy(src_hbm.at[wid], idx_s)
        pltpu.sync_copy(dst_hbm.at[wid], idx_d)
        plsc.subcore_barrier()

        def body(j, carry):
            pltpu.async_copy(f_hbm.at[idx_s.at[j]], rows, sem).wait()
            pltpu.sync_copy(rows, acc.at[idx_d.at[j]], add=True)
            return carry

        @pl.when(c == 0)
        def _c0():
            lax.fori_loop(0, CH0, body, 0)

        @pl.when(c != 0)
        def _c1():
            lax.fori_loop(0, CH1, body, 0)

        plsc.subcore_barrier()
        pltpu.sync_copy(acc.at[pl.ds(s * rpt, rpt)],
                        out_hbm.at[c, pl.ds(s * rpt, rpt)])

    return prop_kernel


def _a_body(deg_ref, x_ref, w_ref, f1_ref, dinv_ref):
    deg = deg_ref[0, :, 0:1] + deg_ref[1, :, 0:1] + 1.0
    dinv = lax.rsqrt(deg)
    xw = jnp.dot(x_ref[...], w_ref[...], preferred_element_type=jnp.float32)
    f1_ref[...] = dinv * xw
    dinv_ref[...] = dinv


def _b_body(p_ref, f1_ref, dinv_ref, b1_ref, w2_ref, f2_ref):
    s = p_ref[0] + p_ref[1] + f1_ref[...]
    h1 = jnp.maximum(dinv_ref[...] * s + b1_ref[...], 0.0)
    f2_ref[...] = dinv_ref[...] * jnp.dot(
        h1, w2_ref[...], preferred_element_type=jnp.float32)


def _c_body(G, C, p_ref, f2_ref, dinv_ref, b2_ref, batch_ref, wc_ref, bc_ref,
            out_ref, sums, cnts):
    i = pl.program_id(0)

    @pl.when(i == 0)
    def _init():
        sums[...] = jnp.zeros_like(sums)
        cnts[...] = jnp.zeros_like(cnts)

    s = p_ref[0] + p_ref[1] + f2_ref[...]
    h2 = jnp.maximum(dinv_ref[...] * s + b2_ref[...], 0.0)
    ids = batch_ref[...]
    seg = lax.broadcasted_iota(jnp.int32, (ids.shape[0], G), 1)
    onehot = jnp.where(ids == seg, 1.0, 0.0)
    dn = (((0,), (0,)), ((), ()))
    sums[...] += lax.dot_general(onehot, h2, dn,
                                 preferred_element_type=jnp.float32)
    ones_col = jnp.ones((ids.shape[0], 1), jnp.float32)
    cnts[...] += lax.dot_general(onehot, ones_col, dn,
                                 preferred_element_type=jnp.float32)

    @pl.when(i == pl.num_programs(0) - 1)
    def _fin():
        pooled = sums[...] / jnp.maximum(cnts[...], 1.0)
        out_ref[...] = jnp.dot(pooled, wc_ref[...],
                               preferred_element_type=jnp.float32) + bc_ref[...]


def kernel(x, edge_index, batch, W1, b1, W2, b2, Wc, bc):
    N, D = x.shape
    H = W1.shape[1]
    C = Wc.shape[1]
    E = edge_index.shape[1]
    G = 64

    NP = ((N + NW * 16 - 1) // (NW * 16)) * (NW * 16)
    egrain = NW * CHUNK
    EP = ((E + egrain - 1) // egrain) * egrain
    CHT = EP // (NS * CHUNK)
    CH0 = (CHT * SPLIT0_NUM) // SPLIT_DEN
    CH1 = CHT - CH0
    CHM = max(CH0, CH1)
    trash = NP - 1

    ei = edge_index.astype(jnp.int32)
    epad = jnp.full((EP - E,), trash, jnp.int32)
    if CH0 == CH1:
        src = jnp.concatenate([ei[0], epad]).reshape(NW, CHM, CHUNK)
        dst = jnp.concatenate([ei[1], epad]).reshape(NW, CHM, CHUNK)
    else:
        flat_s = jnp.concatenate([ei[0], epad])
        flat_d = jnp.concatenate([ei[1], epad])
        src = jnp.full((NW, CHM * CHUNK), trash, jnp.int32)
        dst = jnp.full((NW, CHM * CHUNK), trash, jnp.int32)
        off = 0
        for w in range(NW):
            ln = (CH0 if w % NC == 0 else CH1) * CHUNK
            src = src.at[w, :ln].set(lax.slice(flat_s, (off,), (off + ln,)))
            dst = dst.at[w, :ln].set(lax.slice(flat_d, (off,), (off + ln,)))
            off += ln
        src = src.reshape(NW, CHM, CHUNK)
        dst = dst.reshape(NW, CHM, CHUNK)
    xp = jnp.zeros((NP, D), jnp.float32).at[:N].set(x)
    batch_p = jnp.concatenate(
        [batch.astype(jnp.int32), jnp.full((NP - N,), G, jnp.int32)]
    ).reshape(NP, 1)
    zeros_h = jnp.zeros((NP, H), jnp.float32)
    zeros_1 = jnp.zeros((NP, DW), jnp.float32)
    ones_c = jnp.ones((CHUNK, DW), jnp.float32)

    deg_call = _make_deg(CH0, CH1, CHM, NP)
    prop_call = _make_prop(CH0, CH1, CHM, NP, H)

    deg2 = deg_call(dst, ones_c, zeros_1)

    nblk = NP // ROWBLK
    f1, dinv = pl.pallas_call(
        _a_body,
        grid=(nblk,),
        in_specs=[
            pl.BlockSpec((NC, ROWBLK, DW), lambda i: (0, i, 0)),
            pl.BlockSpec((ROWBLK, D), lambda i: (i, 0)),
            pl.BlockSpec((D, H), lambda i: (0, 0)),
        ],
        out_specs=[
            pl.BlockSpec((ROWBLK, H), lambda i: (i, 0)),
            pl.BlockSpec((ROWBLK, 1), lambda i: (i, 0)),
        ],
        out_shape=[
            jax.ShapeDtypeStruct((NP, H), jnp.float32),
            jax.ShapeDtypeStruct((NP, 1), jnp.float32),
        ],
    )(deg2, xp, W1)

    p1 = prop_call(src, dst, f1, zeros_h)

    f2 = pl.pallas_call(
        _b_body,
        grid=(nblk,),
        in_specs=[
            pl.BlockSpec((NC, ROWBLK, H), lambda i: (0, i, 0)),
            pl.BlockSpec((ROWBLK, H), lambda i: (i, 0)),
            pl.BlockSpec((ROWBLK, 1), lambda i: (i, 0)),
            pl.BlockSpec((1, H), lambda i: (0, 0)),
            pl.BlockSpec((H, H), lambda i: (0, 0)),
        ],
        out_specs=pl.BlockSpec((ROWBLK, H), lambda i: (i, 0)),
        out_shape=jax.ShapeDtypeStruct((NP, H), jnp.float32),
    )(p1, f1, dinv, b1.reshape(1, H), W2)

    p2 = prop_call(src, dst, f2, zeros_h)

    out = pl.pallas_call(
        functools.partial(_c_body, G, C),
        grid=(nblk,),
        in_specs=[
            pl.BlockSpec((NC, ROWBLK, H), lambda i: (0, i, 0)),
            pl.BlockSpec((ROWBLK, H), lambda i: (i, 0)),
            pl.BlockSpec((ROWBLK, 1), lambda i: (i, 0)),
            pl.BlockSpec((1, H), lambda i: (0, 0)),
            pl.BlockSpec((ROWBLK, 1), lambda i: (i, 0)),
            pl.BlockSpec((H, C), lambda i: (0, 0)),
            pl.BlockSpec((1, C), lambda i: (0, 0)),
        ],
        out_specs=pl.BlockSpec((G, C), lambda i: (0, 0)),
        out_shape=jax.ShapeDtypeStruct((G, C), jnp.float32),
        scratch_shapes=[
            pltpu.VMEM((G, H), jnp.float32),
            pltpu.VMEM((G, 1), jnp.float32),
        ],
    )(p2, f2, dinv, b2.reshape(1, H), batch_p, Wc, bc.reshape(1, C))

    return out


# speedup vs baseline: 1.4552x; 1.4552x over previous
import functools

import jax
import jax.numpy as jnp
from jax import lax
from jax.experimental import pallas as pl
from jax.experimental.pallas import tpu as pltpu
from jax.experimental.pallas import tpu_sc as plsc

NC = 2
NS = 16
NW = NC * NS
CHUNK = 128
SPLIT0_NUM, SPLIT_DEN = 5, 10
DW = 16
ROWBLK = 1024


def _sc_mesh():
    return plsc.VectorSubcoreMesh(core_axis_name="c", subcore_axis_name="s")


def _make_deg(CH0, CH1, CHM, NP):
    rpt = NP // NS

    @functools.partial(
        pl.kernel,
        out_type=jax.ShapeDtypeStruct((NC, NP, DW), jnp.float32),
        mesh=_sc_mesh(),
        compiler_params=pltpu.CompilerParams(use_tc_tiling_on_sc=False),
        scratch_types=[
            pltpu.VMEM((CHM, CHUNK), jnp.int32),
            pltpu.VMEM((CHUNK, DW), jnp.float32),
            pltpu.VMEM_SHARED((NP, DW), jnp.float32),
        ],
    )
    def deg_kernel(dst_hbm, ones_hbm, zeros_hbm, out_hbm, idx_d, ones_v, acc):
        c = lax.axis_index("c")
        s = lax.axis_index("s")
        wid = s * NC + c
        pltpu.sync_copy(zeros_hbm.at[pl.ds(s * rpt, rpt)],
                        acc.at[pl.ds(s * rpt, rpt)])
        pltpu.sync_copy(dst_hbm.at[wid], idx_d)
        pltpu.sync_copy(ones_hbm, ones_v)
        plsc.subcore_barrier()

        def body(j, carry):
            pltpu.sync_copy(ones_v, acc.at[idx_d.at[j]], add=True)
            return carry

        @pl.when(c == 0)
        def _c0():
            lax.fori_loop(0, CH0, body, 0)

        @pl.when(c != 0)
        def _c1():
            lax.fori_loop(0, CH1, body, 0)

        plsc.subcore_barrier()
        pltpu.sync_copy(acc.at[pl.ds(s * rpt, rpt)],
                        out_hbm.at[c, pl.ds(s * rpt, rpt)])

    return deg_kernel


def _make_prop(CH0, CH1, CHM, NP, H):
    rpt = NP // NS

    @functools.partial(
        pl.kernel,
        out_type=jax.ShapeDtypeStruct((NC, NP, H), jnp.float32),
        mesh=_sc_mesh(),
        compiler_params=pltpu.CompilerParams(use_tc_tiling_on_sc=False),
        scratch_types=[
            pltpu.VMEM((CHM, CHUNK), jnp.int32),
            pltpu.VMEM((CHM, CHUNK), jnp.int32),
            pltpu.VMEM((CHUNK, H), jnp.float32),
            pltpu.SemaphoreType.DMA,
            pltpu.VMEM_SHARED((NP, H), jnp.float32),
            pltpu.VMEM_SHARED((NP, H), jnp.float32),
        ],
    )
    def prop_kernel(src_hbm, dst_hbm, f_hbm, zeros_hbm, out_hbm,
                    idx_s, idx_d, rows, sem, acc, f_sh):
        c = lax.axis_index("c")
        s = lax.axis_index("s")
        wid = s * NC + c
        pltpu.sync_copy(zeros_hbm.at[pl.ds(s * rpt, rpt)],
                        acc.at[pl.ds(s * rpt, rpt)])
        pltpu.sync_copy(f_hbm.at[pl.ds(s * rpt, rpt)],
                        f_sh.at[pl.ds(s * rpt, rpt)])
        pltpu.sync_copy(src_hbm.at[wid], idx_s)
        pltpu.sync_copy(dst_hbm.at[wid], idx_d)
        plsc.subcore_barrier()

        def body(j, carry):
            pltpu.async_copy(f_sh.at[idx_s.at[j]], rows, sem).wait()
            pltpu.sync_copy(rows, acc.at[idx_d.at[j]], add=True)
            return carry

        @pl.when(c == 0)
        def _c0():
            lax.fori_loop(0, CH0, body, 0)

        @pl.when(c != 0)
        def _c1():
            lax.fori_loop(0, CH1, body, 0)

        plsc.subcore_barrier()
        pltpu.sync_copy(acc.at[pl.ds(s * rpt, rpt)],
                        out_hbm.at[c, pl.ds(s * rpt, rpt)])

    return prop_kernel


def _a_body(deg_ref, x_ref, w_ref, f1_ref, dinv_ref):
    deg = deg_ref[0, :, 0:1] + deg_ref[1, :, 0:1] + 1.0
    dinv = lax.rsqrt(deg)
    xw = jnp.dot(x_ref[...], w_ref[...], preferred_element_type=jnp.float32)
    f1_ref[...] = dinv * xw
    dinv_ref[...] = dinv


def _b_body(p_ref, f1_ref, dinv_ref, b1_ref, w2_ref, f2_ref):
    s = p_ref[0] + p_ref[1] + f1_ref[...]
    h1 = jnp.maximum(dinv_ref[...] * s + b1_ref[...], 0.0)
    f2_ref[...] = dinv_ref[...] * jnp.dot(
        h1, w2_ref[...], preferred_element_type=jnp.float32)


def _c_body(G, C, p_ref, f2_ref, dinv_ref, b2_ref, batch_ref, wc_ref, bc_ref,
            out_ref, sums, cnts):
    i = pl.program_id(0)

    @pl.when(i == 0)
    def _init():
        sums[...] = jnp.zeros_like(sums)
        cnts[...] = jnp.zeros_like(cnts)

    s = p_ref[0] + p_ref[1] + f2_ref[...]
    h2 = jnp.maximum(dinv_ref[...] * s + b2_ref[...], 0.0)
    ids = batch_ref[...]
    seg = lax.broadcasted_iota(jnp.int32, (ids.shape[0], G), 1)
    onehot = jnp.where(ids == seg, 1.0, 0.0)
    dn = (((0,), (0,)), ((), ()))
    sums[...] += lax.dot_general(onehot, h2, dn,
                                 preferred_element_type=jnp.float32)
    ones_col = jnp.ones((ids.shape[0], 1), jnp.float32)
    cnts[...] += lax.dot_general(onehot, ones_col, dn,
                                 preferred_element_type=jnp.float32)

    @pl.when(i == pl.num_programs(0) - 1)
    def _fin():
        pooled = sums[...] / jnp.maximum(cnts[...], 1.0)
        out_ref[...] = jnp.dot(pooled, wc_ref[...],
                               preferred_element_type=jnp.float32) + bc_ref[...]


def kernel(x, edge_index, batch, W1, b1, W2, b2, Wc, bc):
    N, D = x.shape
    H = W1.shape[1]
    C = Wc.shape[1]
    E = edge_index.shape[1]
    G = 64

    NP = ((N + NW * 16 - 1) // (NW * 16)) * (NW * 16)
    egrain = NW * CHUNK
    EP = ((E + egrain - 1) // egrain) * egrain
    CHT = EP // (NS * CHUNK)
    CH0 = (CHT * SPLIT0_NUM) // SPLIT_DEN
    CH1 = CHT - CH0
    CHM = max(CH0, CH1)
    trash = NP - 1

    ei = edge_index.astype(jnp.int32)
    epad = jnp.full((EP - E,), trash, jnp.int32)
    if CH0 == CH1:
        src = jnp.concatenate([ei[0], epad]).reshape(NW, CHM, CHUNK)
        dst = jnp.concatenate([ei[1], epad]).reshape(NW, CHM, CHUNK)
    else:
        flat_s = jnp.concatenate([ei[0], epad])
        flat_d = jnp.concatenate([ei[1], epad])
        src = jnp.full((NW, CHM * CHUNK), trash, jnp.int32)
        dst = jnp.full((NW, CHM * CHUNK), trash, jnp.int32)
        off = 0
        for w in range(NW):
            ln = (CH0 if w % NC == 0 else CH1) * CHUNK
            src = src.at[w, :ln].set(lax.slice(flat_s, (off,), (off + ln,)))
            dst = dst.at[w, :ln].set(lax.slice(flat_d, (off,), (off + ln,)))
            off += ln
        src = src.reshape(NW, CHM, CHUNK)
        dst = dst.reshape(NW, CHM, CHUNK)
    xp = jnp.zeros((NP, D), jnp.float32).at[:N].set(x)
    batch_p = jnp.concatenate(
        [batch.astype(jnp.int32), jnp.full((NP - N,), G, jnp.int32)]
    ).reshape(NP, 1)
    zeros_h = jnp.zeros((NP, H), jnp.float32)
    zeros_1 = jnp.zeros((NP, DW), jnp.float32)
    ones_c = jnp.ones((CHUNK, DW), jnp.float32)

    deg_call = _make_deg(CH0, CH1, CHM, NP)
    prop_call = _make_prop(CH0, CH1, CHM, NP, H)

    deg2 = deg_call(dst, ones_c, zeros_1)

    nblk = NP // ROWBLK
    f1, dinv = pl.pallas_call(
        _a_body,
        grid=(nblk,),
        in_specs=[
            pl.BlockSpec((NC, ROWBLK, DW), lambda i: (0, i, 0)),
            pl.BlockSpec((ROWBLK, D), lambda i: (i, 0)),
            pl.BlockSpec((D, H), lambda i: (0, 0)),
        ],
        out_specs=[
            pl.BlockSpec((ROWBLK, H), lambda i: (i, 0)),
            pl.BlockSpec((ROWBLK, 1), lambda i: (i, 0)),
        ],
        out_shape=[
            jax.ShapeDtypeStruct((NP, H), jnp.float32),
            jax.ShapeDtypeStruct((NP, 1), jnp.float32),
        ],
    )(deg2, xp, W1)

    p1 = prop_call(src, dst, f1, zeros_h)

    f2 = pl.pallas_call(
        _b_body,
        grid=(nblk,),
        in_specs=[
            pl.BlockSpec((NC, ROWBLK, H), lambda i: (0, i, 0)),
            pl.BlockSpec((ROWBLK, H), lambda i: (i, 0)),
            pl.BlockSpec((ROWBLK, 1), lambda i: (i, 0)),
            pl.BlockSpec((1, H), lambda i: (0, 0)),
            pl.BlockSpec((H, H), lambda i: (0, 0)),
        ],
        out_specs=pl.BlockSpec((ROWBLK, H), lambda i: (i, 0)),
        out_shape=jax.ShapeDtypeStruct((NP, H), jnp.float32),
    )(p1, f1, dinv, b1.reshape(1, H), W2)

    p2 = prop_call(src, dst, f2, zeros_h)

    out = pl.pallas_call(
        functools.partial(_c_body, G, C),
        grid=(nblk,),
        in_specs=[
            pl.BlockSpec((NC, ROWBLK, H), lambda i: (0, i, 0)),
            pl.BlockSpec((ROWBLK, H), lambda i: (i, 0)),
            pl.BlockSpec((ROWBLK, 1), lambda i: (i, 0)),
            pl.BlockSpec((1, H), lambda i: (0, 0)),
            pl.BlockSpec((ROWBLK, 1), lambda i: (i, 0)),
            pl.BlockSpec((H, C), lambda i: (0, 0)),
            pl.BlockSpec((1, C), lambda i: (0, 0)),
        ],
        out_specs=pl.BlockSpec((G, C), lambda i: (0, 0)),
        out_shape=jax.ShapeDtypeStruct((G, C), jnp.float32),
        scratch_shapes=[
            pltpu.VMEM((G, H), jnp.float32),
            pltpu.VMEM((G, 1), jnp.float32),
        ],
    )(p2, f2, dinv, b2.reshape(1, H), batch_p, Wc, bc.reshape(1, C))

    return out
